# baseline (speedup 1.0000x reference)
import functools

import jax
import jax.numpy as jnp
from jax import lax
from jax.experimental import pallas as pl
from jax.experimental.pallas import tpu as pltpu
from jax.experimental.pallas import tpu_sc as plsc

HIDDEN = 768
NC, NS = 2, 16
NW = NC * NS
B, S = 4, 2048
NTOK = B * S
TPW = NTOK // NW
C = 64
NCH = TPW // C
NSLOT = 2
BT = 2048
EPS = 1e-12

_mesh = plsc.VectorSubcoreMesh(
    core_axis_name="c", subcore_axis_name="s", num_cores=NC, num_subcores=NS)


@functools.partial(
    pl.kernel,
    out_type=jax.ShapeDtypeStruct((NTOK, HIDDEN), jnp.float32),
    mesh=_mesh,
    compiler_params=pltpu.CompilerParams(needs_layout_passes=False),
    scratch_types=[
        pltpu.VMEM((TPW,), jnp.int32),
        pltpu.VMEM((NSLOT, C, HIDDEN), jnp.float32),
    ] + [pltpu.SemaphoreType.DMA] * (2 * NSLOT),
)
def _sc_gather(ids_hbm, word_hbm, out_hbm, idx_v, rows_v, *sems):
    gsems = sems[:NSLOT]
    osems = sems[NSLOT:]
    wid = lax.axis_index("s") * NC + lax.axis_index("c")
    base = pl.multiple_of(wid * TPW, TPW)

    pltpu.sync_copy(ids_hbm.at[pl.ds(base, TPW)], idx_v)

    def gather(c, slot):
        return pltpu.make_async_copy(
            word_hbm.at[idx_v.at[pl.ds(c * C, C)]], rows_v.at[slot],
            gsems[slot])

    def put(c, slot):
        return pltpu.make_async_copy(
            rows_v.at[slot], out_hbm.at[pl.ds(base + c * C, C)], osems[slot])

    for c in range(NSLOT):
        gather(c, c).start()
    for c in range(NCH):
        slot = c % NSLOT
        gather(c, slot).wait()
        put(c, slot).start()
        nxt = c + NSLOT
        if nxt < NCH:
            put(c, slot).wait()
            gather(nxt, slot).start()
    for c in range(NCH - NSLOT, NCH):
        put(c, c % NSLOT).wait()


def _ln_body(w_ref, p_ref, t_ref, g_ref, b_ref, o_ref):
    v = w_ref[0] + p_ref[...] + t_ref[...]
    mean = jnp.mean(v, axis=-1, keepdims=True)
    d = v - mean
    var = jnp.mean(d * d, axis=-1, keepdims=True)
    o_ref[0] = d * lax.rsqrt(var + EPS) * g_ref[...] + b_ref[...]


def _tc_layernorm(rows3, pos, typ2, gamma2, beta2):
    grid = (S // BT, B)
    return pl.pallas_call(
        _ln_body,
        grid=grid,
        in_specs=[
            pl.BlockSpec((1, BT, HIDDEN), lambda j, b: (b, j, 0)),
            pl.BlockSpec((BT, HIDDEN), lambda j, b: (j, 0)),
            pl.BlockSpec((1, HIDDEN), lambda j, b: (0, 0)),
            pl.BlockSpec((1, HIDDEN), lambda j, b: (0, 0)),
            pl.BlockSpec((1, HIDDEN), lambda j, b: (0, 0)),
        ],
        out_specs=pl.BlockSpec((1, BT, HIDDEN), lambda j, b: (b, j, 0)),
        out_shape=jax.ShapeDtypeStruct((B, S, HIDDEN), jnp.float32),
        compiler_params=pltpu.CompilerParams(
            dimension_semantics=("arbitrary", "arbitrary")),
    )(rows3, pos, typ2, gamma2, beta2)


def kernel(input_ids, word_emb, type_emb, pos_emb, gamma, beta):
    b, s = input_ids.shape
    ids = input_ids.reshape(-1).astype(jnp.int32)
    rows = _sc_gather(ids, word_emb)
    return _tc_layernorm(rows.reshape(b, s, HIDDEN), pos_emb,
                         type_emb[0].reshape(1, HIDDEN),
                         gamma.reshape(1, HIDDEN), beta.reshape(1, HIDDEN))


# speedup vs baseline: 1.0226x; 1.0226x over previous
import functools

import jax
import jax.numpy as jnp
from jax import lax
from jax.experimental import pallas as pl
from jax.experimental.pallas import tpu as pltpu
from jax.experimental.pallas import tpu_sc as plsc

HIDDEN = 768
NC, NS = 2, 16
NW = NC * NS
B, S = 4, 2048
NTOK = B * S
TPW = NTOK // NW
C = 16
NCH = TPW // C
NSLOT = 8
BT = 2048
EPS = 1e-12

_mesh = plsc.VectorSubcoreMesh(
    core_axis_name="c", subcore_axis_name="s", num_cores=NC, num_subcores=NS)


@functools.partial(
    pl.kernel,
    out_type=jax.ShapeDtypeStruct((NTOK, HIDDEN), jnp.float32),
    mesh=_mesh,
    compiler_params=pltpu.CompilerParams(needs_layout_passes=False),
    scratch_types=[
        pltpu.VMEM((TPW,), jnp.int32),
        pltpu.VMEM((NSLOT, C, HIDDEN), jnp.float32),
    ] + [pltpu.SemaphoreType.DMA] * (2 * NSLOT),
)
def _sc_gather(ids_hbm, word_hbm, out_hbm, idx_v, rows_v, *sems):
    gsems = sems[:NSLOT]
    osems = sems[NSLOT:]
    wid = lax.axis_index("s") * NC + lax.axis_index("c")
    base = pl.multiple_of(wid * TPW, TPW)

    pltpu.sync_copy(ids_hbm.at[pl.ds(base, TPW)], idx_v)

    def gather(c, slot):
        return pltpu.make_async_copy(
            word_hbm.at[idx_v.at[pl.ds(c * C, C)]], rows_v.at[slot],
            gsems[slot])

    def put(c, slot):
        return pltpu.make_async_copy(
            rows_v.at[slot], out_hbm.at[pl.ds(base + c * C, C)], osems[slot])

    for c in range(NSLOT):
        gather(c, c).start()
    for c in range(NCH):
        slot = c % NSLOT
        gather(c, slot).wait()
        put(c, slot).start()
        nxt = c + NSLOT
        if nxt < NCH:
            put(c, slot).wait()
            gather(nxt, slot).start()
    for c in range(NCH - NSLOT, NCH):
        put(c, c % NSLOT).wait()


def _ln_body(w_ref, p_ref, t_ref, g_ref, b_ref, o_ref):
    v = w_ref[0] + p_ref[...] + t_ref[...]
    mean = jnp.mean(v, axis=-1, keepdims=True)
    d = v - mean
    var = jnp.mean(d * d, axis=-1, keepdims=True)
    o_ref[0] = d * lax.rsqrt(var + EPS) * g_ref[...] + b_ref[...]


def _tc_layernorm(rows3, pos, typ2, gamma2, beta2):
    grid = (S // BT, B)
    return pl.pallas_call(
        _ln_body,
        grid=grid,
        in_specs=[
            pl.BlockSpec((1, BT, HIDDEN), lambda j, b: (b, j, 0)),
            pl.BlockSpec((BT, HIDDEN), lambda j, b: (j, 0)),
            pl.BlockSpec((1, HIDDEN), lambda j, b: (0, 0)),
            pl.BlockSpec((1, HIDDEN), lambda j, b: (0, 0)),
            pl.BlockSpec((1, HIDDEN), lambda j, b: (0, 0)),
        ],
        out_specs=pl.BlockSpec((1, BT, HIDDEN), lambda j, b: (b, j, 0)),
        out_shape=jax.ShapeDtypeStruct((B, S, HIDDEN), jnp.float32),
        compiler_params=pltpu.CompilerParams(
            dimension_semantics=("arbitrary", "arbitrary")),
    )(rows3, pos, typ2, gamma2, beta2)


def kernel(input_ids, word_emb, type_emb, pos_emb, gamma, beta):
    b, s = input_ids.shape
    ids = input_ids.reshape(-1).astype(jnp.int32)
    rows = _sc_gather(ids, word_emb)
    return _tc_layernorm(rows.reshape(b, s, HIDDEN), pos_emb,
                         type_emb[0].reshape(1, HIDDEN),
                         gamma.reshape(1, HIDDEN), beta.reshape(1, HIDDEN))
